# baseline (speedup 1.0000x reference)
import functools
import math

import numpy as np
import jax
import jax.numpy as jnp
from jax import lax
from jax.experimental import pallas as pl
from jax.experimental.pallas import tpu as pltpu
from jax.experimental.pallas import tpu_sc as plsc

_SIGN = np.array(0x80000000, dtype=np.uint32).view(np.int32).item()
_BITS32 = [np.array(1 << b, dtype=np.uint32).view(np.int32).item()
           for b in range(32)]
_NC, _NS, _L = 2, 16, 16
_NW = _NC * _NS


def _sim_body(inv_scale, x_ref, p_ref, o_ref, pbf_ref):
    @pl.when(pl.program_id(0) == 0)
    def _prep():
        pbf_ref[...] = p_ref[...].astype(jnp.bfloat16)

    xb = x_ref[...].astype(jnp.bfloat16)
    o_ref[...] = jax.lax.dot_general(
        pbf_ref[...], xb, (((1,), (1,)), ((), ())),
        preferred_element_type=jnp.float32) * inv_scale


def _skey(v):
    ki = jax.lax.bitcast_convert_type(v, jnp.int32)
    return jnp.where(ki >= 0, ki, jnp.int32(_SIGN) - ki)


def _select_body(ngrp_w, nconc, nsel, simT_hbm, kki_hbm, thr_hbm,
                 slab, thrbuf, kkv):
    wid = lax.axis_index("s") * _NC + lax.axis_index("c")
    g0 = wid * ngrp_w
    pltpu.sync_copy(kki_hbm, kkv)
    kki = kkv[...]

    def grp_step(g, _):
        t0 = (g0 + g) * _L
        pltpu.sync_copy(simT_hbm.at[:, pl.ds(t0, _L)], slab)

        def key_step(n, _):
            kv = _skey(slab[n])
            slab[n] = jax.lax.bitcast_convert_type(kv, jnp.float32)
            return 0
        lax.fori_loop(0, nconc, key_step, 0)

        t = jnp.zeros((_L,), jnp.int32)
        for b in range(31, 31 - nsel, -1):
            cand = jnp.bitwise_or(t, jnp.int32(_BITS32[b]))
            scand = jnp.bitwise_xor(cand, jnp.int32(_SIGN))

            def cnt_step(n, acc, scand=scand):
                kv = jax.lax.bitcast_convert_type(slab[n], jnp.int32)
                return acc + jnp.where(kv >= scand, 1, 0)
            acc = lax.fori_loop(0, nconc, cnt_step,
                                jnp.zeros((_L,), jnp.int32))
            t = jnp.where(acc >= kki, cand, t)
        thr = jnp.bitwise_xor(t, jnp.int32(_SIGN))
        kithr = jnp.where(thr >= 0, thr, jnp.int32(_SIGN) - thr)
        thrbuf[pl.ds(g * _L, _L)] = jax.lax.bitcast_convert_type(
            kithr, jnp.float32)
        return 0

    lax.fori_loop(0, ngrp_w, grp_step, 0)
    pltpu.sync_copy(thrbuf, thr_hbm.at[pl.ds(g0 * _L, ngrp_w * _L)])


def _tail_body(inv_scale, nrel, rank, g_ref, x_ref, thr_ref, p_ref, a_ref,
               b_ref, o_ref, pbf_ref, acat_ref, bcat_ref):
    @pl.when(pl.program_id(0) == 0)
    def _prep():
        pbf_ref[...] = p_ref[...].astype(jnp.bfloat16)
        for r in range(nrel):
            g = g_ref[r]
            acat_ref[:, r * rank:(r + 1) * rank] = (
                a_ref[r] * g).astype(jnp.bfloat16)
            bcat_ref[:, r * rank:(r + 1) * rank] = (
                b_ref[r]).astype(jnp.bfloat16)

    x = x_ref[...]
    sim = jax.lax.dot_general(
        x.astype(jnp.bfloat16), pbf_ref[...], (((1,), (1,)), ((), ())),
        preferred_element_type=jnp.float32) * inv_scale
    mask = sim >= thr_ref[...]

    rowmax = jnp.max(sim, axis=1, keepdims=True)
    e = jnp.where(mask, jnp.exp(sim - rowmax), 0.0)
    s = jnp.sum(e, axis=1, keepdims=True)
    w = (e / s).astype(jnp.bfloat16)

    z = jnp.dot(w, acat_ref[...], preferred_element_type=jnp.float32)
    c = jax.lax.dot_general(
        z.astype(jnp.bfloat16), bcat_ref[...], (((1,), (1,)), ((), ())),
        preferred_element_type=jnp.float32)
    sm = jnp.where(mask, c, 0.0).astype(jnp.bfloat16)
    ctx = jnp.dot(sm, pbf_ref[...], preferred_element_type=jnp.float32)

    dsum = g_ref[0]
    for r in range(1, nrel):
        dsum = dsum + g_ref[r]
    dsum = jnp.where(dsum <= 0, jnp.float32(1.0), dsum)
    o_ref[...] = x + (jnp.float32(0.1) / dsum) * ctx


def kernel(x, proto, A, Bm, gains, top_k):
    B, T, D = x.shape
    N = proto.shape[0]
    NREL, _, R = A.shape
    kk = jnp.minimum(jnp.asarray(top_k, jnp.int32), min(128, N))

    T2 = B * T
    TB = 128
    while T2 % TB:
        TB //= 2
    x2 = x.reshape(T2, D)

    sim = pl.pallas_call(
        functools.partial(_sim_body, 1.0 / math.sqrt(D)),
        grid=(T2 // TB,),
        in_specs=[
            pl.BlockSpec((TB, D), lambda i: (i, 0)),
            pl.BlockSpec((N, D), lambda i: (0, 0)),
        ],
        out_specs=pl.BlockSpec((N, TB), lambda i: (0, i)),
        out_shape=jax.ShapeDtypeStruct((N, T2), jnp.float32),
        scratch_shapes=[pltpu.VMEM((N, D), jnp.bfloat16)],
        compiler_params=pltpu.CompilerParams(
            dimension_semantics=("arbitrary",)),
    )(x2, proto)

    kki = jnp.broadcast_to(kk, (_L,))
    ngrp_w = T2 // (_NW * _L)
    sel = pl.kernel(
        functools.partial(_select_body, ngrp_w, N, 16),
        out_type=jax.ShapeDtypeStruct((T2,), jnp.float32),
        mesh=plsc.VectorSubcoreMesh(core_axis_name="c",
                                    subcore_axis_name="s"),
        compiler_params=pltpu.CompilerParams(use_tc_tiling_on_sc=False),
        scratch_types=[
            pltpu.VMEM((N, _L), jnp.float32),
            pltpu.VMEM((ngrp_w * _L,), jnp.float32),
            pltpu.VMEM((_L,), jnp.int32),
        ],
    )
    thr = sel(sim, kki)

    TBC = TB
    out = pl.pallas_call(
        functools.partial(_tail_body, 1.0 / math.sqrt(D), NREL, R),
        grid=(T2 // TBC,),
        in_specs=[
            pl.BlockSpec(memory_space=pltpu.SMEM),
            pl.BlockSpec((TBC, D), lambda i: (i, 0)),
            pl.BlockSpec((TBC, 1), lambda i: (i, 0)),
            pl.BlockSpec((N, D), lambda i: (0, 0)),
            pl.BlockSpec((NREL, N, R), lambda i: (0, 0, 0)),
            pl.BlockSpec((NREL, N, R), lambda i: (0, 0, 0)),
        ],
        out_specs=pl.BlockSpec((TBC, D), lambda i: (i, 0)),
        out_shape=jax.ShapeDtypeStruct((T2, D), jnp.float32),
        scratch_shapes=[
            pltpu.VMEM((N, D), jnp.bfloat16),
            pltpu.VMEM((N, NREL * R), jnp.bfloat16),
            pltpu.VMEM((N, NREL * R), jnp.bfloat16),
        ],
        compiler_params=pltpu.CompilerParams(
            dimension_semantics=("arbitrary",)),
    )(gains, x2, thr.reshape(T2, 1), proto, A, Bm)
    return out.reshape(B, T, D)


# speedup vs baseline: 6.5817x; 6.5817x over previous
import functools
import math

import numpy as np
import jax
import jax.numpy as jnp
from jax.experimental import pallas as pl
from jax.experimental.pallas import tpu as pltpu

_SIGN = np.array(0x80000000, dtype=np.uint32).view(np.int32).item()
_BITS32 = [np.array(1 << b, dtype=np.uint32).view(np.int32).item()
           for b in range(32)]
_NSEL = 16


def _body(inv_scale, nrel, rank, kk_ref, g_ref, x_ref, p_ref, a_ref, b_ref,
          o_ref, pbf_ref, acat_ref, bcat_ref):
    @pl.when(pl.program_id(0) == 0)
    def _prep():
        pbf_ref[...] = p_ref[...].astype(jnp.bfloat16)
        for r in range(nrel):
            g = g_ref[r]
            acat_ref[:, r * rank:(r + 1) * rank] = (
                a_ref[r] * g).astype(jnp.bfloat16)
            bcat_ref[:, r * rank:(r + 1) * rank] = (
                b_ref[r]).astype(jnp.bfloat16)

    kk = kk_ref[0]
    x = x_ref[...]
    xb = x.astype(jnp.bfloat16)
    sim = jax.lax.dot_general(
        xb, pbf_ref[...], (((1,), (1,)), ((), ())),
        preferred_element_type=jnp.float32) * inv_scale

    ki = jax.lax.bitcast_convert_type(sim, jnp.int32)
    key = jnp.where(ki >= 0, ki, jnp.int32(_SIGN) - ki)

    t = jnp.zeros((x.shape[0], 1), jnp.int32)
    for b in range(31, 31 - _NSEL, -1):
        cand = jnp.bitwise_or(t, jnp.int32(_BITS32[b]))
        scand = jnp.bitwise_xor(cand, jnp.int32(_SIGN))
        cnt = jnp.sum((key >= scand).astype(jnp.int32), axis=1, keepdims=True)
        t = jnp.where(cnt >= kk, cand, t)
    thr = jnp.bitwise_xor(t, jnp.int32(_SIGN))
    mask = key >= thr

    rowmax = jnp.max(sim, axis=1, keepdims=True)
    e = jnp.where(mask, jnp.exp(sim - rowmax), 0.0)
    s = jnp.sum(e, axis=1, keepdims=True)
    w = (e / s).astype(jnp.bfloat16)

    z = jnp.dot(w, acat_ref[...], preferred_element_type=jnp.float32)
    c = jax.lax.dot_general(
        z.astype(jnp.bfloat16), bcat_ref[...], (((1,), (1,)), ((), ())),
        preferred_element_type=jnp.float32)
    sm = jnp.where(mask, c, 0.0).astype(jnp.bfloat16)
    ctx = jnp.dot(sm, pbf_ref[...], preferred_element_type=jnp.float32)

    dsum = g_ref[0]
    for r in range(1, nrel):
        dsum = dsum + g_ref[r]
    dsum = jnp.where(dsum <= 0, jnp.float32(1.0), dsum)
    o_ref[...] = x + (jnp.float32(0.1) / dsum) * ctx


def kernel(x, proto, A, Bm, gains, top_k):
    B, T, D = x.shape
    N = proto.shape[0]
    NREL, _, R = A.shape
    kk = jnp.minimum(jnp.asarray(top_k, jnp.int32), min(128, N)).reshape(1)

    T2 = B * T
    TB = 128
    while T2 % TB:
        TB //= 2
    x2 = x.reshape(T2, D)

    out = pl.pallas_call(
        functools.partial(_body, 1.0 / math.sqrt(D), NREL, R),
        grid=(T2 // TB,),
        in_specs=[
            pl.BlockSpec(memory_space=pltpu.SMEM),
            pl.BlockSpec(memory_space=pltpu.SMEM),
            pl.BlockSpec((TB, D), lambda i: (i, 0)),
            pl.BlockSpec((N, D), lambda i: (0, 0)),
            pl.BlockSpec((NREL, N, R), lambda i: (0, 0, 0)),
            pl.BlockSpec((NREL, N, R), lambda i: (0, 0, 0)),
        ],
        out_specs=pl.BlockSpec((TB, D), lambda i: (i, 0)),
        out_shape=jax.ShapeDtypeStruct((T2, D), jnp.float32),
        scratch_shapes=[
            pltpu.VMEM((N, D), jnp.bfloat16),
            pltpu.VMEM((N, NREL * R), jnp.bfloat16),
            pltpu.VMEM((N, NREL * R), jnp.bfloat16),
        ],
        compiler_params=pltpu.CompilerParams(
            dimension_semantics=("arbitrary",)),
    )(kk, gains, x2, proto, A, Bm)
    return out.reshape(B, T, D)
